# baseline (speedup 1.0000x reference)
import functools

import jax
import jax.numpy as jnp
from jax import lax
from jax.experimental import pallas as pl
from jax.experimental.pallas import tpu as pltpu
from jax.experimental.pallas import tpu_sc as plsc

N = 10000
E = 160000
F1 = 256
NH = 512

NC = 2
NS = 16
NW = NC * NS
NPW = 320
N_PAD = NW * NPW
CE = 1600
NCE = E // CE
CH = 2048
LISTCAP = E + CH + 16
SC_E = 1024
KG = 32
NCHK = SC_E // KG
NB = 4
CC = 1024
F32 = jnp.float32
I32 = jnp.int32

_MESH = plsc.VectorSubcoreMesh(core_axis_name="c", subcore_axis_name="s")
_CP = pltpu.CompilerParams(needs_layout_passes=False)


def _wid():
    return lax.axis_index("s") * NC + lax.axis_index("c")


def _iota16():
    return lax.iota(I32, 16)


def _zero_vmem(ref, n, dtype):
    z = jnp.zeros((16,), dtype)

    @plsc.parallel_loop(0, n // 16, unroll=4)
    def body(t):
        ref[pl.ds(t * 16, 16)] = z


def _prep_body(row_hbm, col_hbm, ew_hbm,
               dis_hbm, lr_hbm, ld_hbm, lw_hbm, cnt_hbm,
               rowb0, colb0, ewb0, rowb1, colb1, ewb1,
               bufR, bufD, bufW, degv, disv, cntv, esem0, esem1):
    w = _wid()
    n0 = w * NPW
    lbase = w * LISTCAP

    _zero_vmem(degv, NPW, F32)

    bufs = ((rowb0, colb0, ewb0), (rowb1, colb1, ewb1))
    sems = (esem0, esem1)

    def fire(ci, p):
        co = pl.multiple_of(ci * CE, 8)
        rb, cb, eb = bufs[p]
        pltpu.async_copy(row_hbm.at[pl.ds(co, CE)], rb, sems[p])
        pltpu.async_copy(col_hbm.at[pl.ds(co, CE)], cb, sems[p])
        pltpu.async_copy(ew_hbm.at[pl.ds(co, CE)], eb, sems[p])

    def drain(p):
        rb, cb, eb = bufs[p]
        pltpu.make_async_copy(row_hbm.at[pl.ds(0, CE)], rb, sems[p]).wait()
        pltpu.make_async_copy(col_hbm.at[pl.ds(0, CE)], cb, sems[p]).wait()
        pltpu.make_async_copy(ew_hbm.at[pl.ds(0, CE)], eb, sems[p]).wait()

    def flush(outoff, size):
        o = pl.multiple_of(lbase + outoff, 8)
        pltpu.sync_copy(bufR.at[pl.ds(0, size)], lr_hbm.at[pl.ds(o, size)])
        pltpu.sync_copy(bufD.at[pl.ds(0, size)], ld_hbm.at[pl.ds(o, size)])
        pltpu.sync_copy(bufW.at[pl.ds(0, size)], lw_hbm.at[pl.ds(o, size)])

    def process(p, carry):
        rb, cb, eb = bufs[p]
        off0, outoff = carry

        @plsc.parallel_loop(0, CE // 16, unroll=2, carry=off0)
        def grp_body(g, off):
            c = cb[pl.ds(g * 16, 16)]
            r = rb[pl.ds(g * 16, 16)]
            e = eb[pl.ds(g * 16, 16)]
            dloc = c - n0
            m = (dloc >= 0) & (dloc < NPW)
            plsc.addupdate_scatter(degv, [jnp.where(m, dloc, 0)],
                                   jnp.where(m, e, 0.0))
            plsc.store_compressed(bufR.at[pl.ds(off, 16)], r, mask=m)
            plsc.store_compressed(bufD.at[pl.ds(off, 16)], dloc, mask=m)
            plsc.store_compressed(bufW.at[pl.ds(off, 16)], e, mask=m)
            pc = plsc.all_reduce_population_count(m)
            pc0 = pc if pc.ndim == 0 else pc[0]
            return off + pc0

        off = grp_body
        do_flush = off >= CH

        @pl.when(do_flush)
        def _():
            flush(outoff, CH)

            def mv(t, _):
                bufR[pl.ds(t * 16, 16)] = bufR[pl.ds(CH + t * 16, 16)]
                bufD[pl.ds(t * 16, 16)] = bufD[pl.ds(CH + t * 16, 16)]
                bufW[pl.ds(t * 16, 16)] = bufW[pl.ds(CH + t * 16, 16)]
                return 0

            lax.fori_loop(0, (CE + 16) // 16, mv, 0)

        off = jnp.where(do_flush, off - CH, off)
        outoff = jnp.where(do_flush, outoff + CH, outoff)
        return off, outoff

    fire(0, 0)

    def pair_body(ci2, carry):
        a = ci2 * 2
        fire(a + 1, 1)
        drain(0)
        carry = process(0, carry)

        @pl.when(a + 2 < NCE)
        def _():
            fire(a + 2, 0)

        drain(1)
        carry = process(1, carry)
        return carry

    off, outoff = lax.fori_loop(0, NCE // 2, pair_body,
                                (jnp.int32(0), jnp.int32(0)))
    flush(outoff, CH + 16)
    total = outoff + off
    cntv[pl.ds(0, 16)] = jnp.full((16,), total, I32)
    pltpu.sync_copy(cntv, cnt_hbm.at[pl.ds(pl.multiple_of(w * 16, 8), 16)])

    def rs_body(t, _):
        d = degv[pl.ds(t * 16, 16)] + 1.0
        i = plsc.bitcast(d, I32)
        i = jnp.int32(0x5F3759DF) - lax.shift_right_logical(i, 1)
        y = plsc.bitcast(i, F32)
        for _ in range(3):
            y = y * (1.5 - 0.5 * d * y * y)
        disv[pl.ds(t * 16, 16)] = y
        return 0

    lax.fori_loop(0, NPW // 16, rs_body, 0)
    pltpu.sync_copy(disv, dis_hbm.at[pl.ds(pl.multiple_of(n0, 8), NPW)])


def _prep(row, col, ew):
    f = pl.kernel(
        _prep_body,
        out_type=(
            jax.ShapeDtypeStruct((N_PAD,), F32),
            jax.ShapeDtypeStruct((NW * LISTCAP,), I32),
            jax.ShapeDtypeStruct((NW * LISTCAP,), I32),
            jax.ShapeDtypeStruct((NW * LISTCAP,), F32),
            jax.ShapeDtypeStruct((NW * 16,), I32),
        ),
        mesh=_MESH,
        compiler_params=_CP,
        scratch_types=[
            pltpu.VMEM((CE,), I32), pltpu.VMEM((CE,), I32),
            pltpu.VMEM((CE,), F32),
            pltpu.VMEM((CE,), I32), pltpu.VMEM((CE,), I32),
            pltpu.VMEM((CE,), F32),
            pltpu.VMEM((CH + CE + 16,), I32),
            pltpu.VMEM((CH + CE + 16,), I32),
            pltpu.VMEM((CH + CE + 16,), F32),
            pltpu.VMEM((NPW,), F32),
            pltpu.VMEM((NPW,), F32),
            pltpu.VMEM((16,), I32),
            pltpu.SemaphoreType.DMA,
            pltpu.SemaphoreType.DMA,
        ],
    )
    return f(row, col, ew)


def _coef_body(lr_hbm, ld_hbm, lw_hbm, cnt_hbm, dis_hbm, cf_hbm,
               rb, db, wb, cb, disv, cntv):
    w = _wid()
    n0 = w * NPW
    lbase = w * LISTCAP

    pltpu.sync_copy(dis_hbm, disv)
    pltpu.sync_copy(cnt_hbm.at[pl.ds(pl.multiple_of(w * 16, 8), 16)], cntv)
    count = jnp.max(cntv[pl.ds(0, 16)])
    nch = (count + CC - 1) // CC

    def ch_body(ci, _):
        base = ci * CC
        lo = pl.multiple_of(lbase + base, 8)
        pltpu.sync_copy(lr_hbm.at[pl.ds(lo, CC)], rb)
        pltpu.sync_copy(ld_hbm.at[pl.ds(lo, CC)], db)
        pltpu.sync_copy(lw_hbm.at[pl.ds(lo, CC)], wb)

        def g_body(g, _):
            sl = pl.ds(g * 16, 16)
            m = (base + g * 16 + _iota16()) < count
            r = jnp.where(m, rb[sl], 0)
            d = jnp.where(m, db[sl], 0)
            e = jnp.where(m, wb[sl], 0.0)
            cb[sl] = e * plsc.load_gather(disv, [r]) \
                       * plsc.load_gather(disv, [d + n0])
            return 0

        lax.fori_loop(0, CC // 16, g_body, 0)
        pltpu.sync_copy(cb, cf_hbm.at[pl.ds(lo, CC)])
        return 0

    lax.fori_loop(0, nch, ch_body, 0)


def _coef(lr, ld, lw, cnts, dis):
    f = pl.kernel(
        _coef_body,
        out_type=jax.ShapeDtypeStruct((NW * LISTCAP,), F32),
        mesh=_MESH,
        compiler_params=_CP,
        scratch_types=[
            pltpu.VMEM((CC,), I32), pltpu.VMEM((CC,), I32),
            pltpu.VMEM((CC,), F32), pltpu.VMEM((CC,), F32),
            pltpu.VMEM((N_PAD,), F32),
            pltpu.VMEM((16,), I32),
        ],
    )
    return f(lr, ld, lw, cnts, dis)


def _agg_body(src_hbm, lr_hbm, ld_hbm, cf_hbm, cnt_hbm, z_hbm,
              ribuf, dibuf, cfbuf, acc,
              rowb0, rowb1, rowb2, rowb3,
              ib0, ib1, ib2, ib3, cntv,
              gsem0, gsem1, gsem2, gsem3):
    w = _wid()
    n0 = w * NPW
    lbase = w * LISTCAP
    rowbs = (rowb0, rowb1, rowb2, rowb3)
    idxbs = (ib0, ib1, ib2, ib3)
    gsems = (gsem0, gsem1, gsem2, gsem3)

    _zero_vmem(acc, NPW * F1, F32)
    pltpu.sync_copy(cnt_hbm.at[pl.ds(pl.multiple_of(w * 16, 8), 16)], cntv)
    count = jnp.max(cntv[pl.ds(0, 16)])
    nsuper = (count + SC_E - 1) // SC_E

    def fire(i, p):
        ib = idxbs[p]

        @plsc.parallel_loop(0, KG // 16, unroll=2)
        def _cp(t):
            ib[pl.ds(t * 16, 16)] = ribuf[pl.ds(i * KG + t * 16, 16)]

        pltpu.async_copy(src_hbm.at[ib], rowbs[p], gsems[p])

    def drain(p):
        pltpu.make_async_copy(src_hbm.at[pl.ds(0, KG)], rowbs[p],
                              gsems[p]).wait()

    def acc_chunk(i, p):
        rowb = rowbs[p]
        iota = _iota16()
        j0 = i * KG

        @plsc.parallel_loop(0, KG, unroll=4)
        def edge(j):
            esplat = jnp.full((16,), j0 + j, I32)
            cfb = plsc.load_gather(cfbuf, [esplat])
            db = plsc.load_gather(dibuf, [esplat])
            idx0 = db * F1 + iota
            for k in range(F1 // 16):
                v = rowb[j, pl.ds(k * 16, 16)]
                plsc.addupdate_scatter(acc, [idx0 + (k * 16)], v * cfb)

    def super_body(s, _):
        base = s * SC_E
        lo = pl.multiple_of(lbase + base, 8)
        pltpu.sync_copy(lr_hbm.at[pl.ds(lo, SC_E)], ribuf)
        pltpu.sync_copy(ld_hbm.at[pl.ds(lo, SC_E)], dibuf)
        pltpu.sync_copy(cf_hbm.at[pl.ds(lo, SC_E)], cfbuf)

        @plsc.parallel_loop(0, SC_E // 16, unroll=2)
        def mask_body(g):
            sl = pl.ds(g * 16, 16)
            m = (base + g * 16 + _iota16()) < count
            ribuf[sl] = jnp.where(m, ribuf[sl], 0)
            dibuf[sl] = jnp.where(m, dibuf[sl], 0)
            cfbuf[sl] = jnp.where(m, cfbuf[sl], 0.0)

        for t in range(NB - 1):
            fire(t, t)

        def ring_outer(i0, _):
            for t in range(NB):
                c = i0 * NB + t

                @pl.when(c + (NB - 1) < NCHK)
                def _():
                    fire(c + (NB - 1), (t + NB - 1) % NB)

                drain(t)
                acc_chunk(c, t)
            return 0

        lax.fori_loop(0, NCHK // NB, ring_outer, 0)
        return 0

    lax.fori_loop(0, nsuper, super_body, 0)
    pltpu.sync_copy(acc, z_hbm.at[pl.ds(pl.multiple_of(n0 * F1, 8),
                                        NPW * F1)])


def _make_agg():
    return pl.kernel(
        _agg_body,
        out_type=jax.ShapeDtypeStruct((N_PAD * F1,), F32),
        mesh=_MESH,
        compiler_params=_CP,
        scratch_types=[
            pltpu.VMEM((SC_E,), I32),
            pltpu.VMEM((SC_E,), I32),
            pltpu.VMEM((SC_E,), F32),
            pltpu.VMEM((NPW * F1,), F32),
            pltpu.VMEM((KG, F1), F32),
            pltpu.VMEM((KG, F1), F32),
            pltpu.VMEM((KG, F1), F32),
            pltpu.VMEM((KG, F1), F32),
            pltpu.VMEM((KG,), I32),
            pltpu.VMEM((KG,), I32),
            pltpu.VMEM((KG,), I32),
            pltpu.VMEM((KG,), I32),
            pltpu.VMEM((16,), I32),
            pltpu.SemaphoreType.DMA,
            pltpu.SemaphoreType.DMA,
            pltpu.SemaphoreType.DMA,
            pltpu.SemaphoreType.DMA,
        ],
    )


_agg = _make_agg()


_MB = 1024


def _tc1_body(z_ref, x_ref, dis_ref, w_ref, b_ref, h0_ref, h1_ref):
    dis = dis_ref[...]
    t = z_ref[...] + (dis * dis) * x_ref[...]
    h = jnp.dot(t, w_ref[...], preferred_element_type=F32) + b_ref[...]
    h = jnp.maximum(h, 0.0)
    h0_ref[...] = h[:, :F1]
    h1_ref[...] = h[:, F1:]


def _tc1(z1, x, dis, W1, b1):
    grid = (pl.cdiv(N, _MB),)
    return pl.pallas_call(
        _tc1_body,
        grid=grid,
        in_specs=[
            pl.BlockSpec((_MB, F1), lambda i: (i, 0)),
            pl.BlockSpec((_MB, F1), lambda i: (i, 0)),
            pl.BlockSpec((_MB, 1), lambda i: (i, 0)),
            pl.BlockSpec((F1, NH), lambda i: (0, 0)),
            pl.BlockSpec((1, NH), lambda i: (0, 0)),
        ],
        out_specs=[
            pl.BlockSpec((_MB, F1), lambda i: (i, 0)),
            pl.BlockSpec((_MB, F1), lambda i: (i, 0)),
        ],
        out_shape=[
            jax.ShapeDtypeStruct((N, F1), F32),
            jax.ShapeDtypeStruct((N, F1), F32),
        ],
    )(z1, x, dis, W1, b1.reshape(1, NH))


def _tc2_body(z0_ref, z1_ref, h0_ref, h1_ref, dis_ref, w0_ref, w1_ref, b_ref,
              o_ref):
    dis = dis_ref[...]
    d2 = dis * dis
    t0 = z0_ref[...] + d2 * h0_ref[...]
    t1 = z1_ref[...] + d2 * h1_ref[...]
    o = (jnp.dot(t0, w0_ref[...], preferred_element_type=F32)
         + jnp.dot(t1, w1_ref[...], preferred_element_type=F32)
         + b_ref[...])
    o_ref[...] = jnp.maximum(o, 0.0)


def _tc2(z0, z1, h0, h1, dis, W2, b2):
    grid = (pl.cdiv(N, _MB),)
    return pl.pallas_call(
        _tc2_body,
        grid=grid,
        in_specs=[
            pl.BlockSpec((_MB, F1), lambda i: (i, 0)),
            pl.BlockSpec((_MB, F1), lambda i: (i, 0)),
            pl.BlockSpec((_MB, F1), lambda i: (i, 0)),
            pl.BlockSpec((_MB, F1), lambda i: (i, 0)),
            pl.BlockSpec((_MB, 1), lambda i: (i, 0)),
            pl.BlockSpec((F1, NH), lambda i: (0, 0)),
            pl.BlockSpec((F1, NH), lambda i: (0, 0)),
            pl.BlockSpec((1, NH), lambda i: (0, 0)),
        ],
        out_specs=pl.BlockSpec((_MB, NH), lambda i: (i, 0)),
        out_shape=jax.ShapeDtypeStruct((N, NH), F32),
    )(z0, z1, h0, h1, dis, W2[:F1], W2[F1:], b2.reshape(1, NH))


def kernel(x, edge_index, edge_weight, W1, b1, W2, b2):
    row = edge_index[0].astype(I32)
    col = edge_index[1].astype(I32)
    dis_p, lr, ld, lw, cnts = _prep(row, col, edge_weight)
    coefl = _coef(lr, ld, lw, cnts, dis_p)
    z1 = _agg(x, lr, ld, coefl, cnts).reshape(N_PAD, F1)[:N]
    dis = dis_p[:N].reshape(N, 1)
    h0, h1 = _tc1(z1, x, dis, W1, b1)
    z2a = _agg(h0, lr, ld, coefl, cnts).reshape(N_PAD, F1)[:N]
    z2b = _agg(h1, lr, ld, coefl, cnts).reshape(N_PAD, F1)[:N]
    return _tc2(z2a, z2b, h0, h1, dis, W2, b2)


# speedup vs baseline: 1.4809x; 1.4809x over previous
import jax
import jax.numpy as jnp
from jax import lax
from jax.experimental import pallas as pl
from jax.experimental.pallas import tpu as pltpu
from jax.experimental.pallas import tpu_sc as plsc

N = 10000
E = 160000
F1 = 256
NH = 512

NC = 2
NS = 16
NW = NC * NS
NPW = 320
N_PAD = NW * NPW
CE = 1600
NCE = E // CE
CH = 2048
LISTCAP = E + CH + 16
SC_E = 1024
KG = 32
NCHK = SC_E // KG
NB = 4
CC = 1024
KG2 = 16
NPH = NPW // 2
F32 = jnp.float32
I32 = jnp.int32

_MESH = plsc.VectorSubcoreMesh(core_axis_name="c", subcore_axis_name="s")
_CP = pltpu.CompilerParams(needs_layout_passes=False)


def _wid():
    return lax.axis_index("s") * NC + lax.axis_index("c")


def _iota16():
    return lax.iota(I32, 16)


def _zero_vmem(ref, n, dtype):
    z = jnp.zeros((16,), dtype)

    @plsc.parallel_loop(0, n // 16, unroll=4)
    def body(t):
        ref[pl.ds(t * 16, 16)] = z


def _prep_body(row_hbm, col_hbm, ew_hbm,
               dis_hbm, lr_hbm, ld_hbm, lw_hbm, cnt_hbm,
               rowb0, colb0, ewb0, rowb1, colb1, ewb1,
               bufR, bufD, bufW, degv, disv, cntv, esem0, esem1):
    w = _wid()
    n0 = w * NPW
    lbase = w * LISTCAP

    _zero_vmem(degv, NPW, F32)

    bufs = ((rowb0, colb0, ewb0), (rowb1, colb1, ewb1))
    sems = (esem0, esem1)

    def fire(ci, p):
        co = pl.multiple_of(ci * CE, 8)
        rb, cb, eb = bufs[p]
        pltpu.async_copy(row_hbm.at[pl.ds(co, CE)], rb, sems[p])
        pltpu.async_copy(col_hbm.at[pl.ds(co, CE)], cb, sems[p])
        pltpu.async_copy(ew_hbm.at[pl.ds(co, CE)], eb, sems[p])

    def drain(p):
        rb, cb, eb = bufs[p]
        pltpu.make_async_copy(row_hbm.at[pl.ds(0, CE)], rb, sems[p]).wait()
        pltpu.make_async_copy(col_hbm.at[pl.ds(0, CE)], cb, sems[p]).wait()
        pltpu.make_async_copy(ew_hbm.at[pl.ds(0, CE)], eb, sems[p]).wait()

    def flush(outoff, size):
        o = pl.multiple_of(lbase + outoff, 8)
        pltpu.sync_copy(bufR.at[pl.ds(0, size)], lr_hbm.at[pl.ds(o, size)])
        pltpu.sync_copy(bufD.at[pl.ds(0, size)], ld_hbm.at[pl.ds(o, size)])
        pltpu.sync_copy(bufW.at[pl.ds(0, size)], lw_hbm.at[pl.ds(o, size)])

    def process(p, carry):
        rb, cb, eb = bufs[p]
        off0, outoff = carry

        @plsc.parallel_loop(0, CE // 16, unroll=2, carry=off0)
        def grp_body(g, off):
            c = cb[pl.ds(g * 16, 16)]
            r = rb[pl.ds(g * 16, 16)]
            e = eb[pl.ds(g * 16, 16)]
            dloc = c - n0
            m = (dloc >= 0) & (dloc < NPW)
            plsc.addupdate_scatter(degv, [jnp.where(m, dloc, 0)],
                                   jnp.where(m, e, 0.0))
            plsc.store_compressed(bufR.at[pl.ds(off, 16)], r, mask=m)
            plsc.store_compressed(bufD.at[pl.ds(off, 16)], dloc, mask=m)
            plsc.store_compressed(bufW.at[pl.ds(off, 16)], e, mask=m)
            pc = plsc.all_reduce_population_count(m)
            pc0 = pc if pc.ndim == 0 else pc[0]
            return off + pc0

        off = grp_body
        do_flush = off >= CH

        @pl.when(do_flush)
        def _():
            flush(outoff, CH)

            def mv(t, _):
                bufR[pl.ds(t * 16, 16)] = bufR[pl.ds(CH + t * 16, 16)]
                bufD[pl.ds(t * 16, 16)] = bufD[pl.ds(CH + t * 16, 16)]
                bufW[pl.ds(t * 16, 16)] = bufW[pl.ds(CH + t * 16, 16)]
                return 0

            lax.fori_loop(0, (CE + 16) // 16, mv, 0)

        off = jnp.where(do_flush, off - CH, off)
        outoff = jnp.where(do_flush, outoff + CH, outoff)
        return off, outoff

    fire(0, 0)

    def pair_body(ci2, carry):
        a = ci2 * 2
        fire(a + 1, 1)
        drain(0)
        carry = process(0, carry)

        @pl.when(a + 2 < NCE)
        def _():
            fire(a + 2, 0)

        drain(1)
        carry = process(1, carry)
        return carry

    off, outoff = lax.fori_loop(0, NCE // 2, pair_body,
                                (jnp.int32(0), jnp.int32(0)))
    flush(outoff, CH + 16)
    total = outoff + off
    cntv[pl.ds(0, 16)] = jnp.full((16,), total, I32)
    pltpu.sync_copy(cntv, cnt_hbm.at[pl.ds(pl.multiple_of(w * 16, 8), 16)])

    def rs_body(t, _):
        d = degv[pl.ds(t * 16, 16)] + 1.0
        i = plsc.bitcast(d, I32)
        i = jnp.int32(0x5F3759DF) - lax.shift_right_logical(i, 1)
        y = plsc.bitcast(i, F32)
        for _ in range(3):
            y = y * (1.5 - 0.5 * d * y * y)
        disv[pl.ds(t * 16, 16)] = y
        return 0

    lax.fori_loop(0, NPW // 16, rs_body, 0)
    pltpu.sync_copy(disv, dis_hbm.at[pl.ds(pl.multiple_of(n0, 8), NPW)])


def _prep(row, col, ew):
    f = pl.kernel(
        _prep_body,
        out_type=(
            jax.ShapeDtypeStruct((N_PAD,), F32),
            jax.ShapeDtypeStruct((NW * LISTCAP,), I32),
            jax.ShapeDtypeStruct((NW * LISTCAP,), I32),
            jax.ShapeDtypeStruct((NW * LISTCAP,), F32),
            jax.ShapeDtypeStruct((NW * 16,), I32),
        ),
        mesh=_MESH,
        compiler_params=_CP,
        scratch_types=[
            pltpu.VMEM((CE,), I32), pltpu.VMEM((CE,), I32),
            pltpu.VMEM((CE,), F32),
            pltpu.VMEM((CE,), I32), pltpu.VMEM((CE,), I32),
            pltpu.VMEM((CE,), F32),
            pltpu.VMEM((CH + CE + 16,), I32),
            pltpu.VMEM((CH + CE + 16,), I32),
            pltpu.VMEM((CH + CE + 16,), F32),
            pltpu.VMEM((NPW,), F32),
            pltpu.VMEM((NPW,), F32),
            pltpu.VMEM((16,), I32),
            pltpu.SemaphoreType.DMA,
            pltpu.SemaphoreType.DMA,
        ],
    )
    return f(row, col, ew)


def _coef_body(lr_hbm, ld_hbm, lw_hbm, cnt_hbm, dis_hbm, cf_hbm,
               rb, db, wb, cb, disv, cntv):
    w = _wid()
    n0 = w * NPW
    lbase = w * LISTCAP

    pltpu.sync_copy(dis_hbm, disv)
    pltpu.sync_copy(cnt_hbm.at[pl.ds(pl.multiple_of(w * 16, 8), 16)], cntv)
    count = jnp.max(cntv[pl.ds(0, 16)])
    nch = (count + CC - 1) // CC

    def ch_body(ci, _):
        base = ci * CC
        lo = pl.multiple_of(lbase + base, 8)
        pltpu.sync_copy(lr_hbm.at[pl.ds(lo, CC)], rb)
        pltpu.sync_copy(ld_hbm.at[pl.ds(lo, CC)], db)
        pltpu.sync_copy(lw_hbm.at[pl.ds(lo, CC)], wb)

        def g_body(g, _):
            sl = pl.ds(g * 16, 16)
            m = (base + g * 16 + _iota16()) < count
            r = jnp.where(m, rb[sl], 0)
            d = jnp.where(m, db[sl], 0)
            e = jnp.where(m, wb[sl], 0.0)
            cb[sl] = e * plsc.load_gather(disv, [r]) \
                       * plsc.load_gather(disv, [d + n0])
            return 0

        lax.fori_loop(0, CC // 16, g_body, 0)
        pltpu.sync_copy(cb, cf_hbm.at[pl.ds(lo, CC)])
        return 0

    lax.fori_loop(0, nch, ch_body, 0)


def _coef(lr, ld, lw, cnts, dis):
    f = pl.kernel(
        _coef_body,
        out_type=jax.ShapeDtypeStruct((NW * LISTCAP,), F32),
        mesh=_MESH,
        compiler_params=_CP,
        scratch_types=[
            pltpu.VMEM((CC,), I32), pltpu.VMEM((CC,), I32),
            pltpu.VMEM((CC,), F32), pltpu.VMEM((CC,), F32),
            pltpu.VMEM((N_PAD,), F32),
            pltpu.VMEM((16,), I32),
        ],
    )
    return f(lr, ld, lw, cnts, dis)


def _agg_body(src_hbm, lr_hbm, ld_hbm, cf_hbm, cnt_hbm, z_hbm,
              ribuf, dibuf, cfbuf, acc,
              rowb0, rowb1, rowb2, rowb3,
              ib0, ib1, ib2, ib3, cntv,
              gsem0, gsem1, gsem2, gsem3):
    w = _wid()
    n0 = w * NPW
    lbase = w * LISTCAP
    rowbs = (rowb0, rowb1, rowb2, rowb3)
    idxbs = (ib0, ib1, ib2, ib3)
    gsems = (gsem0, gsem1, gsem2, gsem3)

    _zero_vmem(acc, NPW * F1, F32)
    pltpu.sync_copy(cnt_hbm.at[pl.ds(pl.multiple_of(w * 16, 8), 16)], cntv)
    count = jnp.max(cntv[pl.ds(0, 16)])
    nsuper = (count + SC_E - 1) // SC_E

    def fire(i, p):
        ib = idxbs[p]

        @plsc.parallel_loop(0, KG // 16, unroll=2)
        def _cp(t):
            ib[pl.ds(t * 16, 16)] = ribuf[pl.ds(i * KG + t * 16, 16)]

        pltpu.async_copy(src_hbm.at[ib], rowbs[p], gsems[p])

    def drain(p):
        pltpu.make_async_copy(src_hbm.at[pl.ds(0, KG)], rowbs[p],
                              gsems[p]).wait()

    def acc_chunk(i, p):
        rowb = rowbs[p]
        iota = _iota16()
        j0 = i * KG

        @plsc.parallel_loop(0, KG, unroll=4)
        def edge(j):
            esplat = jnp.full((16,), j0 + j, I32)
            cfb = plsc.load_gather(cfbuf, [esplat])
            db = plsc.load_gather(dibuf, [esplat])
            idx0 = db * F1 + iota
            for k in range(F1 // 16):
                v = rowb[j, pl.ds(k * 16, 16)]
                plsc.addupdate_scatter(acc, [idx0 + (k * 16)], v * cfb)

    def super_body(s, _):
        base = s * SC_E
        lo = pl.multiple_of(lbase + base, 8)
        pltpu.sync_copy(lr_hbm.at[pl.ds(lo, SC_E)], ribuf)
        pltpu.sync_copy(ld_hbm.at[pl.ds(lo, SC_E)], dibuf)
        pltpu.sync_copy(cf_hbm.at[pl.ds(lo, SC_E)], cfbuf)

        @plsc.parallel_loop(0, SC_E // 16, unroll=2)
        def mask_body(g):
            sl = pl.ds(g * 16, 16)
            m = (base + g * 16 + _iota16()) < count
            ribuf[sl] = jnp.where(m, ribuf[sl], 0)
            dibuf[sl] = jnp.where(m, dibuf[sl], 0)
            cfbuf[sl] = jnp.where(m, cfbuf[sl], 0.0)

        for t in range(NB - 1):
            fire(t, t)

        def ring_outer(i0, _):
            for t in range(NB):
                c = i0 * NB + t

                @pl.when(c + (NB - 1) < NCHK)
                def _():
                    fire(c + (NB - 1), (t + NB - 1) % NB)

                drain(t)
                acc_chunk(c, t)
            return 0

        lax.fori_loop(0, NCHK // NB, ring_outer, 0)
        return 0

    lax.fori_loop(0, nsuper, super_body, 0)
    pltpu.sync_copy(acc, z_hbm.at[pl.ds(pl.multiple_of(n0 * F1, 8),
                                        NPW * F1)])


def _make_agg():
    return pl.kernel(
        _agg_body,
        out_type=jax.ShapeDtypeStruct((N_PAD * F1,), F32),
        mesh=_MESH,
        compiler_params=_CP,
        scratch_types=[
            pltpu.VMEM((SC_E,), I32),
            pltpu.VMEM((SC_E,), I32),
            pltpu.VMEM((SC_E,), F32),
            pltpu.VMEM((NPW * F1,), F32),
            pltpu.VMEM((KG, F1), F32),
            pltpu.VMEM((KG, F1), F32),
            pltpu.VMEM((KG, F1), F32),
            pltpu.VMEM((KG, F1), F32),
            pltpu.VMEM((KG,), I32),
            pltpu.VMEM((KG,), I32),
            pltpu.VMEM((KG,), I32),
            pltpu.VMEM((KG,), I32),
            pltpu.VMEM((16,), I32),
            pltpu.SemaphoreType.DMA,
            pltpu.SemaphoreType.DMA,
            pltpu.SemaphoreType.DMA,
            pltpu.SemaphoreType.DMA,
        ],
    )


_agg = _make_agg()


def _agg2_body(src_hbm, lr_hbm, ld_hbm, cf_hbm, cnt_hbm, z_hbm,
               ribuf, dibuf, cfbuf, crb, cdb, ccb, acc,
               rowb0, rowb1, rowb2, rowb3,
               ib0, ib1, ib2, ib3, cntv,
               gsem0, gsem1, gsem2, gsem3):
    w = _wid()
    n0 = w * NPW
    lbase = w * LISTCAP
    rowbs = (rowb0, rowb1, rowb2, rowb3)
    idxbs = (ib0, ib1, ib2, ib3)
    gsems = (gsem0, gsem1, gsem2, gsem3)
    iota = _iota16()

    pltpu.sync_copy(cnt_hbm.at[pl.ds(pl.multiple_of(w * 16, 8), 16)], cntv)
    count = jnp.max(cntv[pl.ds(0, 16)])
    nsuper = (count + SC_E - 1) // SC_E

    def fire(i, p):
        ib = idxbs[p]
        ib[pl.ds(0, 16)] = crb[pl.ds(i * KG2, 16)]
        pltpu.async_copy(src_hbm.at[ib], rowbs[p], gsems[p])

    def drain(p):
        pltpu.make_async_copy(src_hbm.at[pl.ds(0, KG2)], rowbs[p],
                              gsems[p]).wait()

    def acc_chunk(i, p):
        rowb = rowbs[p]
        j0 = i * KG2

        @plsc.parallel_loop(0, KG2, unroll=4)
        def edge(j):
            esplat = jnp.full((16,), j0 + j, I32)
            cfb = plsc.load_gather(ccb, [esplat])
            db = plsc.load_gather(cdb, [esplat])
            idx0 = db * NH + iota
            for k in range(NH // 16):
                v = rowb[j, pl.ds(k * 16, 16)]
                plsc.addupdate_scatter(acc, [idx0 + (k * 16)], v * cfb)

    for half in range(2):
        h0 = half * NPH
        _zero_vmem(acc, NPH * NH, F32)

        def super_body(s, _):
            base = s * SC_E
            lo = pl.multiple_of(lbase + base, 8)
            pltpu.sync_copy(lr_hbm.at[pl.ds(lo, SC_E)], ribuf)
            pltpu.sync_copy(ld_hbm.at[pl.ds(lo, SC_E)], dibuf)
            pltpu.sync_copy(cf_hbm.at[pl.ds(lo, SC_E)], cfbuf)

            def filt(g, coff):
                sl = pl.ds(g * 16, 16)
                d = dibuf[sl] - h0
                m = ((base + g * 16 + iota) < count)                     & (d >= 0) & (d < NPH)
                plsc.store_compressed(crb.at[pl.ds(coff, 16)],
                                      ribuf[sl], mask=m)
                plsc.store_compressed(cdb.at[pl.ds(coff, 16)], d, mask=m)
                plsc.store_compressed(ccb.at[pl.ds(coff, 16)],
                                      cfbuf[sl], mask=m)
                pc = plsc.all_reduce_population_count(m)
                pc0 = pc if pc.ndim == 0 else pc[0]
                return coff + pc0

            coff = lax.fori_loop(0, SC_E // 16, filt, jnp.int32(0))
            crb[pl.ds(coff, 16)] = jnp.zeros((16,), I32)
            cdb[pl.ds(coff, 16)] = jnp.zeros((16,), I32)
            ccb[pl.ds(coff, 16)] = jnp.zeros((16,), F32)
            nck = (coff + KG2 - 1) // KG2

            for t in range(NB - 1):
                @pl.when(t < nck)
                def _():
                    fire(t, t)

            def ring_outer(i0, _):
                for t in range(NB):
                    c = i0 * NB + t

                    @pl.when(c < nck)
                    def _():
                        @pl.when(c + (NB - 1) < nck)
                        def _():
                            fire(c + (NB - 1), (t + NB - 1) % NB)

                        drain(t)
                        acc_chunk(c, t)
                return 0

            lax.fori_loop(0, ((SC_E // KG2) + NB - 1) // NB, ring_outer, 0)
            return 0

        lax.fori_loop(0, nsuper, super_body, 0)
        pltpu.sync_copy(
            acc, z_hbm.at[pl.ds(pl.multiple_of((n0 + h0) * NH, 8), NPH * NH)])


def _make_agg2():
    return pl.kernel(
        _agg2_body,
        out_type=jax.ShapeDtypeStruct((N_PAD * NH,), F32),
        mesh=_MESH,
        compiler_params=_CP,
        scratch_types=[
            pltpu.VMEM((SC_E,), I32),
            pltpu.VMEM((SC_E,), I32),
            pltpu.VMEM((SC_E,), F32),
            pltpu.VMEM((SC_E + 16,), I32),
            pltpu.VMEM((SC_E + 16,), I32),
            pltpu.VMEM((SC_E + 16,), F32),
            pltpu.VMEM((NPH * NH,), F32),
            pltpu.VMEM((KG2, NH), F32),
            pltpu.VMEM((KG2, NH), F32),
            pltpu.VMEM((KG2, NH), F32),
            pltpu.VMEM((KG2, NH), F32),
            pltpu.VMEM((16,), I32),
            pltpu.VMEM((16,), I32),
            pltpu.VMEM((16,), I32),
            pltpu.VMEM((16,), I32),
            pltpu.VMEM((16,), I32),
            pltpu.SemaphoreType.DMA,
            pltpu.SemaphoreType.DMA,
            pltpu.SemaphoreType.DMA,
            pltpu.SemaphoreType.DMA,
        ],
    )


_agg2 = _make_agg2()


_MB = 1024


def _tc1_body(z_ref, x_ref, dis_ref, w_ref, b_ref, h_ref):
    dis = dis_ref[...]
    t = z_ref[...] + (dis * dis) * x_ref[...]
    h = jnp.dot(t, w_ref[...], preferred_element_type=F32) + b_ref[...]
    h_ref[...] = jnp.maximum(h, 0.0)


def _tc1(z1, x, dis, W1, b1):
    grid = (pl.cdiv(N, _MB),)
    return pl.pallas_call(
        _tc1_body,
        grid=grid,
        in_specs=[
            pl.BlockSpec((_MB, F1), lambda i: (i, 0)),
            pl.BlockSpec((_MB, F1), lambda i: (i, 0)),
            pl.BlockSpec((_MB, 1), lambda i: (i, 0)),
            pl.BlockSpec((F1, NH), lambda i: (0, 0)),
            pl.BlockSpec((1, NH), lambda i: (0, 0)),
        ],
        out_specs=pl.BlockSpec((_MB, NH), lambda i: (i, 0)),
        out_shape=jax.ShapeDtypeStruct((N, NH), F32),
    )(z1, x, dis, W1, b1.reshape(1, NH))


def _tc2_body(z_ref, h_ref, dis_ref, w_ref, b_ref, o_ref):
    dis = dis_ref[...]
    t = z_ref[...] + (dis * dis) * h_ref[...]
    o = jnp.dot(t, w_ref[...], preferred_element_type=F32) + b_ref[...]
    o_ref[...] = jnp.maximum(o, 0.0)


def _tc2(z2, h, dis, W2, b2):
    grid = (pl.cdiv(N, _MB),)
    return pl.pallas_call(
        _tc2_body,
        grid=grid,
        in_specs=[
            pl.BlockSpec((_MB, NH), lambda i: (i, 0)),
            pl.BlockSpec((_MB, NH), lambda i: (i, 0)),
            pl.BlockSpec((_MB, 1), lambda i: (i, 0)),
            pl.BlockSpec((NH, NH), lambda i: (0, 0)),
            pl.BlockSpec((1, NH), lambda i: (0, 0)),
        ],
        out_specs=pl.BlockSpec((_MB, NH), lambda i: (i, 0)),
        out_shape=jax.ShapeDtypeStruct((N, NH), F32),
    )(z2, h, dis, W2, b2.reshape(1, NH))


def kernel(x, edge_index, edge_weight, W1, b1, W2, b2):
    row = edge_index[0].astype(I32)
    col = edge_index[1].astype(I32)
    dis_p, lr, ld, lw, cnts = _prep(row, col, edge_weight)
    coefl = _coef(lr, ld, lw, cnts, dis_p)
    z1 = _agg(x, lr, ld, coefl, cnts).reshape(N_PAD, F1)[:N]
    dis = dis_p[:N].reshape(N, 1)
    h = _tc1(z1, x, dis, W1, b1)
    z2 = _agg2(h, lr, ld, coefl, cnts).reshape(N_PAD, NH)[:N]
    return _tc2(z2, h, dis, W2, b2)
